# baseline (speedup 1.0000x reference)
import functools

import jax
import jax.numpy as jnp
from jax import lax
from jax.experimental import pallas as pl
from jax.experimental.pallas import tpu as pltpu
from jax.experimental.pallas import tpu_sc as plsc

_F32 = jnp.float32


def _ln(u, g, b):
    mu = jnp.mean(u, axis=-1, keepdims=True)
    var = jnp.mean((u - mu) ** 2, axis=-1, keepdims=True)
    return (u - mu) * lax.rsqrt(var + 1e-5) * g + b


def _dot(a, w):
    return jnp.dot(a, w, preferred_element_type=_F32)


def _mlp3(x, Ws, bs, ln, blk):
    M, Din = x.shape
    Dout = Ws[2].shape[1]
    grid = M // blk
    has_ln = ln is not None

    def body(x_ref, w1_ref, b1_ref, w2_ref, b2_ref, w3_ref, b3_ref, *rest):
        if has_ln:
            g_ref, be_ref, o_ref = rest
        else:
            (o_ref,) = rest
        h = jnp.maximum(_dot(x_ref[...], w1_ref[...]) + b1_ref[...], 0.0)
        h = jnp.maximum(_dot(h, w2_ref[...]) + b2_ref[...], 0.0)
        u = _dot(h, w3_ref[...]) + b3_ref[...]
        if has_ln:
            u = _ln(u, g_ref[...], be_ref[...])
        o_ref[...] = u

    full = lambda s: pl.BlockSpec(s, lambda i: (0, 0))
    in_specs = [
        pl.BlockSpec((blk, Din), lambda i: (i, 0)),
        full(Ws[0].shape), full((1, Ws[0].shape[1])),
        full(Ws[1].shape), full((1, Ws[1].shape[1])),
        full(Ws[2].shape), full((1, Dout)),
    ]
    args = [x, Ws[0], bs[0].reshape(1, -1), Ws[1], bs[1].reshape(1, -1),
            Ws[2], bs[2].reshape(1, -1)]
    if has_ln:
        in_specs += [full((1, Dout)), full((1, Dout))]
        args += [ln[0].reshape(1, -1), ln[1].reshape(1, -1)]
    return pl.pallas_call(
        body,
        grid=(grid,),
        in_specs=in_specs,
        out_specs=pl.BlockSpec((blk, Dout), lambda i: (i, 0)),
        out_shape=jax.ShapeDtypeStruct((M, Dout), _F32),
    )(*args)


def _pq(x, w0a, w0b, b0, blk):
    M, D = x.shape
    H = w0a.shape[1]
    grid = M // blk

    def body(x_ref, wa_ref, b_ref, wb_ref, p_ref, q_ref):
        xv = x_ref[...]
        p_ref[...] = _dot(xv, wa_ref[...]) + b_ref[...]
        q_ref[...] = _dot(xv, wb_ref[...])

    full = lambda s: pl.BlockSpec(s, lambda i: (0, 0))
    return pl.pallas_call(
        body,
        grid=(grid,),
        in_specs=[pl.BlockSpec((blk, D), lambda i: (i, 0)),
                  full((D, H)), full((1, H)), full((D, H))],
        out_specs=[pl.BlockSpec((blk, H), lambda i: (i, 0)),
                   pl.BlockSpec((blk, H), lambda i: (i, 0))],
        out_shape=[jax.ShapeDtypeStruct((M, H), _F32),
                   jax.ShapeDtypeStruct((M, H), _F32)],
    )(x, w0a, b0.reshape(1, -1), w0b)


def _edge_pass(xsum, e, w0c, w1, b1, w2, b2, g, be, blk):
    M, H = e.shape
    grid = M // blk

    def body(xsum_ref, e_ref, w0c_ref, w1_ref, b1_ref, w2_ref, b2_ref,
             g_ref, be_ref, o_ref):
        ev = e_ref[...]
        h = jnp.maximum(xsum_ref[...] + _dot(ev, w0c_ref[...]), 0.0)
        h = jnp.maximum(_dot(h, w1_ref[...]) + b1_ref[...], 0.0)
        u = _dot(h, w2_ref[...]) + b2_ref[...]
        o_ref[...] = _ln(u, g_ref[...], be_ref[...]) + ev

    full = lambda s: pl.BlockSpec(s, lambda i: (0, 0))
    row = pl.BlockSpec((blk, H), lambda i: (i, 0))
    return pl.pallas_call(
        body,
        grid=(grid,),
        in_specs=[row, row, full((H, H)), full((H, H)), full((1, H)),
                  full((H, H)), full((1, H)), full((1, H)), full((1, H))],
        out_specs=row,
        out_shape=jax.ShapeDtypeStruct((M, H), _F32),
    )(xsum, e, w0c, w1, b1.reshape(1, -1), w2, b2.reshape(1, -1),
      g.reshape(1, -1), be.reshape(1, -1))


def _node_pass(x, p0, p1, c0, c1, v0a, v0b, cb0, v1, cb1, v2, cb2, g, be, blk):
    M, H = x.shape
    grid = M // blk
    CW = c0.shape[1]

    def body(x_ref, p0_ref, p1_ref, c0_ref, c1_ref, va_ref, vb_ref, b0_ref,
             v1_ref, b1_ref, v2_ref, b2_ref, g_ref, be_ref, o_ref):
        xv = x_ref[...]
        aggr = p0_ref[...] + p1_ref[...]
        h = jnp.maximum(_dot(xv, va_ref[...]) + _dot(aggr, vb_ref[...])
                        + b0_ref[...], 0.0)
        h = jnp.maximum(_dot(h, v1_ref[...]) + b1_ref[...], 0.0)
        u = _dot(h, v2_ref[...]) + b2_ref[...]
        xu = _ln(u, g_ref[...], be_ref[...]) + xv
        cnt = jnp.sum(c0_ref[...] + c1_ref[...], axis=1, keepdims=True)
        o_ref[...] = jnp.where(cnt > 0.0, xu, xv)

    full = lambda s: pl.BlockSpec(s, lambda i: (0, 0))
    row = pl.BlockSpec((blk, H), lambda i: (i, 0))
    crow = pl.BlockSpec((blk, CW), lambda i: (i, 0))
    return pl.pallas_call(
        body,
        grid=(grid,),
        in_specs=[row, row, row, crow, crow, full((H, H)), full((H, H)),
                  full((1, H)), full((H, H)), full((1, H)), full((H, H)),
                  full((1, H)), full((1, H)), full((1, H))],
        out_specs=row,
        out_shape=jax.ShapeDtypeStruct((M, H), _F32),
    )(x, p0, p1, c0, c1, v0a, v0b, cb0.reshape(1, -1), v1, cb1.reshape(1, -1),
      v2, cb2.reshape(1, -1), g.reshape(1, -1), be.reshape(1, -1))


def _make_gather_sum(NT, D, E_pad, NC, NS):
    NW = NC * NS
    EPW = E_pad // NW
    CH = EPW // 128
    mesh = plsc.VectorSubcoreMesh(core_axis_name="c", subcore_axis_name="s")

    @functools.partial(
        pl.kernel, mesh=mesh,
        out_type=jax.ShapeDtypeStruct((E_pad, D), _F32),
        scratch_types=[pltpu.VMEM((CH, 128), jnp.int32),
                       pltpu.VMEM((CH, 128), jnp.int32),
                       pltpu.VMEM((128, D), _F32),
                       pltpu.VMEM((128, D), _F32),
                       pltpu.SemaphoreType.DMA,
                       pltpu.SemaphoreType.DMA,
                       pltpu.SemaphoreType.DMA,
                       pltpu.SemaphoreType.DMA,
                       pltpu.SemaphoreType.DMA,
                       pltpu.SemaphoreType.DMA])
    def gather_sum(p_hbm, q_hbm, dst_hbm, src_hbm, xsum_out,
                   dv, sv, ra0, ra1, sa0, sb0, sa1, sb1, sw0, sw1):
        c = lax.axis_index("c")
        s = lax.axis_index("s")
        wid = s * NC + c
        rbase = wid * CH
        ebase = wid * EPW
        pltpu.sync_copy(dst_hbm.at[pl.ds(rbase, CH)], dv)
        pltpu.sync_copy(src_hbm.at[pl.ds(rbase, CH)], sv)

        def body(i, carry):
            j0 = 2 * i
            j1 = j0 + 1
            cp_a0 = pltpu.async_copy(p_hbm.at[dv.at[j0]], ra0, sa0)
            cp_a1 = pltpu.async_copy(p_hbm.at[dv.at[j1]], ra1, sa1)
            cp_a0.wait()
            cp_b0 = pltpu.async_copy(q_hbm.at[sv.at[j0]], ra0, sb0, add=True)
            cp_a1.wait()
            cp_b1 = pltpu.async_copy(q_hbm.at[sv.at[j1]], ra1, sb1, add=True)
            cp_b0.wait()
            w0 = pltpu.async_copy(
                ra0, xsum_out.at[pl.ds(ebase + j0 * 128, 128)], sw0)
            cp_b1.wait()
            w1 = pltpu.async_copy(
                ra1, xsum_out.at[pl.ds(ebase + j1 * 128, 128)], sw1)
            w0.wait()
            w1.wait()
            return carry

        lax.fori_loop(0, CH // 2, body, 0)

    return gather_sum


def _make_scatter(NP, D, CW, E_pad, NC, NS):
    NW = NC * NS
    EPW = E_pad // NW
    CH = EPW // 128
    NPT = NP // NS
    mesh = plsc.VectorSubcoreMesh(core_axis_name="c", subcore_axis_name="s")

    _chunks = []
    _off = 0
    while _off < NPT:
        _w = min(128, NPT - _off)
        _chunks.append((_off, _w))
        _off += _w

    @functools.partial(
        pl.kernel, mesh=mesh,
        out_type=jax.ShapeDtypeStruct((NC * NP, D), _F32),
        scratch_types=[pltpu.VMEM((CH, 128), jnp.int32),
                       pltpu.VMEM((128, D), _F32),
                       pltpu.VMEM((128, D), _F32),
                       pltpu.VMEM_SHARED((NP, D), _F32),
                       pltpu.SemaphoreType.DMA,
                       pltpu.SemaphoreType.DMA])
    def scatter(vals_hbm, dst_hbm, zd_hbm, part_out,
                dv, vv, vv1, aggr_sh, s0, s1):
        c = lax.axis_index("c")
        s = lax.axis_index("s")
        wid = s * NC + c
        rb = s * NPT
        ob = c * NP
        pltpu.sync_copy(zd_hbm.at[pl.ds(0, 128)], vv)
        for off, w in _chunks:
            pltpu.sync_copy(vv.at[pl.ds(0, w)],
                            aggr_sh.at[pl.ds(rb + off, w)])
        pltpu.sync_copy(dst_hbm.at[pl.ds(wid * CH, CH)], dv)
        plsc.subcore_barrier()

        def body(i, carry):
            j0 = 2 * i
            j1 = j0 + 1
            c0 = pltpu.async_copy(
                vals_hbm.at[pl.ds(wid * EPW + j0 * 128, 128)], vv, s0)
            c1 = pltpu.async_copy(
                vals_hbm.at[pl.ds(wid * EPW + j1 * 128, 128)], vv1, s1)
            c0.wait()
            pltpu.sync_copy(vv, aggr_sh.at[dv.at[j0]], add=True)
            c1.wait()
            pltpu.sync_copy(vv1, aggr_sh.at[dv.at[j1]], add=True)
            return carry

        lax.fori_loop(0, CH // 2, body, 0)
        plsc.subcore_barrier()
        for off, w in _chunks:
            pltpu.sync_copy(aggr_sh.at[pl.ds(rb + off, w)],
                            vv.at[pl.ds(0, w)])
            pltpu.sync_copy(vv.at[pl.ds(0, w)],
                            part_out.at[pl.ds(ob + rb + off, w)])

    return scatter


def _make_count(NP, D, E_pad, NC, NS):
    NW = NC * NS
    EPW = E_pad // NW
    CH = EPW // 128
    NPT = NP // NS
    mesh = plsc.VectorSubcoreMesh(core_axis_name="c", subcore_axis_name="s")

    _chunks = []
    _off = 0
    while _off < NPT:
        _w = min(128, NPT - _off)
        _chunks.append((_off, _w))
        _off += _w

    @functools.partial(
        pl.kernel, mesh=mesh,
        out_type=jax.ShapeDtypeStruct((NC * NP, D), _F32),
        scratch_types=[pltpu.VMEM((1, 128), jnp.int32),
                       pltpu.VMEM((128, D), _F32),
                       pltpu.VMEM_SHARED((NP, D), _F32)])
    def count(dst_hbm, zd_hbm, ones_hbm, cnt_out, dv, vv, cnt_sh):
        c = lax.axis_index("c")
        s = lax.axis_index("s")
        wid = s * NC + c
        rb = s * NPT
        ob = c * NP
        pltpu.sync_copy(zd_hbm.at[pl.ds(0, 128)], vv)
        for off, w in _chunks:
            pltpu.sync_copy(vv.at[pl.ds(0, w)],
                            cnt_sh.at[pl.ds(rb + off, w)])
        pltpu.sync_copy(ones_hbm, vv)
        plsc.subcore_barrier()

        def body(j, carry):
            pltpu.sync_copy(dst_hbm.at[pl.ds(wid * CH + j, 1)], dv)
            pltpu.sync_copy(vv, cnt_sh.at[dv.at[0]], add=True)
            return carry

        lax.fori_loop(0, CH, body, 0)
        plsc.subcore_barrier()
        for off, w in _chunks:
            pltpu.sync_copy(cnt_sh.at[pl.ds(rb + off, w)],
                            vv.at[pl.ds(0, w)])
            pltpu.sync_copy(vv.at[pl.ds(0, w)],
                            cnt_out.at[pl.ds(ob + rb + off, w)])

    return count


def kernel(x, edge_index, edge_attr, params):
    N, D = x.shape
    E = edge_attr.shape[0]
    H = params["enc_node"]["Ws"][2].shape[1]
    CW = 16

    info = plsc.get_sparse_core_info()
    NC, NS = info.num_cores, info.num_subcores
    NW = NC * NS
    E_pad = ((E + NW * 256 - 1) // (NW * 256)) * (NW * 256)
    NP = ((N + 1 + 8 * NS - 1) // (8 * NS)) * (8 * NS)

    src = edge_index[0].astype(jnp.int32)
    dst = edge_index[1].astype(jnp.int32)
    pad = E_pad - E
    dst_g = jnp.concatenate([dst, jnp.zeros((pad,), jnp.int32)]).reshape(-1, 128)
    src_g = jnp.concatenate([src, jnp.zeros((pad,), jnp.int32)]).reshape(-1, 128)
    dst_s = jnp.concatenate([dst, jnp.full((pad,), N, jnp.int32)]).reshape(-1, 128)
    ea_pad = jnp.concatenate(
        [edge_attr, jnp.zeros((pad, edge_attr.shape[1]), _F32)])
    zd = jnp.zeros((NP, H), _F32)
    ones_h = jnp.ones((128, H), _F32)

    gather_sum = _make_gather_sum(N, H, E_pad, NC, NS)
    scatter = _make_scatter(NP, H, CW, E_pad, NC, NS)
    count = _make_count(NP, H, E_pad, NC, NS)

    NBLK = 1000
    EBLK = 1024

    enc_n = params["enc_node"]
    xh = _mlp3(x, enc_n["Ws"], enc_n["bs"], enc_n["ln"], NBLK)
    enc_e = params["enc_edge"]
    e = _mlp3(ea_pad, enc_e["Ws"], enc_e["bs"], enc_e["ln"], EBLK)

    cnts = count(dst_s, zd, ones_h).reshape(NC, NP, H)
    c0, c1 = cnts[0, :N, :CW], cnts[1, :N, :CW]

    for layer in params["proc"]:
        em = layer["edge_mlp"]
        W0 = em["Ws"][0]
        P, Q = _pq(xh, W0[:H], W0[H:2 * H], em["bs"][0], NBLK)
        xsum = gather_sum(P, Q, dst_g, src_g)
        e = _edge_pass(xsum, e, W0[2 * H:], em["Ws"][1], em["bs"][1],
                       em["Ws"][2], em["bs"][2], em["ln"][0], em["ln"][1], EBLK)
        parts = scatter(e, dst_s, zd).reshape(NC, NP, H)
        nm = layer["node_mlp"]
        V0 = nm["Ws"][0]
        xh = _node_pass(xh, parts[0, :N], parts[1, :N], c0, c1,
                        V0[:H], V0[H:], nm["bs"][0], nm["Ws"][1],
                        nm["bs"][1], nm["Ws"][2], nm["bs"][2], nm["ln"][0],
                        nm["ln"][1], NBLK)

    dn = params["dec_node"]
    on = dn["Ws"][2].shape[1]
    w3n = jnp.pad(dn["Ws"][2], ((0, 0), (0, 8 - on)))
    b3n = jnp.pad(dn["bs"][2], (0, 8 - on))
    node_out = _mlp3(x=xh, Ws=[dn["Ws"][0], dn["Ws"][1], w3n],
                     bs=[dn["bs"][0], dn["bs"][1], b3n], ln=None, blk=NBLK)
    de = params["dec_edge"]
    oe = de["Ws"][2].shape[1]
    w3e = jnp.pad(de["Ws"][2], ((0, 0), (0, 8 - oe)))
    b3e = jnp.pad(de["bs"][2], (0, 8 - oe))
    edge_out = _mlp3(x=e, Ws=[de["Ws"][0], de["Ws"][1], w3e],
                     bs=[de["bs"][0], de["bs"][1], b3e], ln=None, blk=EBLK)
    return (node_out[:, :on], edge_out[:E, :oe])


# speedup vs baseline: 1.1829x; 1.1829x over previous
import functools

import jax
import jax.numpy as jnp
from jax import lax
from jax.experimental import pallas as pl
from jax.experimental.pallas import tpu as pltpu
from jax.experimental.pallas import tpu_sc as plsc

_F32 = jnp.float32


def _ln(u, g, b):
    mu = jnp.mean(u, axis=-1, keepdims=True)
    var = jnp.mean((u - mu) ** 2, axis=-1, keepdims=True)
    return (u - mu) * lax.rsqrt(var + 1e-5) * g + b


def _dot(a, w):
    return jnp.dot(a, w, preferred_element_type=_F32)


def _mlp3(x, Ws, bs, ln, blk):
    M, Din = x.shape
    Dout = Ws[2].shape[1]
    grid = M // blk
    has_ln = ln is not None

    def body(x_ref, w1_ref, b1_ref, w2_ref, b2_ref, w3_ref, b3_ref, *rest):
        if has_ln:
            g_ref, be_ref, o_ref = rest
        else:
            (o_ref,) = rest
        h = jnp.maximum(_dot(x_ref[...], w1_ref[...]) + b1_ref[...], 0.0)
        h = jnp.maximum(_dot(h, w2_ref[...]) + b2_ref[...], 0.0)
        u = _dot(h, w3_ref[...]) + b3_ref[...]
        if has_ln:
            u = _ln(u, g_ref[...], be_ref[...])
        o_ref[...] = u

    full = lambda s: pl.BlockSpec(s, lambda i: (0, 0))
    in_specs = [
        pl.BlockSpec((blk, Din), lambda i: (i, 0)),
        full(Ws[0].shape), full((1, Ws[0].shape[1])),
        full(Ws[1].shape), full((1, Ws[1].shape[1])),
        full(Ws[2].shape), full((1, Dout)),
    ]
    args = [x, Ws[0], bs[0].reshape(1, -1), Ws[1], bs[1].reshape(1, -1),
            Ws[2], bs[2].reshape(1, -1)]
    if has_ln:
        in_specs += [full((1, Dout)), full((1, Dout))]
        args += [ln[0].reshape(1, -1), ln[1].reshape(1, -1)]
    return pl.pallas_call(
        body,
        grid=(grid,),
        in_specs=in_specs,
        out_specs=pl.BlockSpec((blk, Dout), lambda i: (i, 0)),
        out_shape=jax.ShapeDtypeStruct((M, Dout), _F32),
    )(*args)


def _pq(x, w0a, w0b, b0, blk):
    M, D = x.shape
    H = w0a.shape[1]
    grid = M // blk

    def body(x_ref, wa_ref, b_ref, wb_ref, p_ref, q_ref):
        xv = x_ref[...]
        p_ref[...] = _dot(xv, wa_ref[...]) + b_ref[...]
        q_ref[...] = _dot(xv, wb_ref[...])

    full = lambda s: pl.BlockSpec(s, lambda i: (0, 0))
    return pl.pallas_call(
        body,
        grid=(grid,),
        in_specs=[pl.BlockSpec((blk, D), lambda i: (i, 0)),
                  full((D, H)), full((1, H)), full((D, H))],
        out_specs=[pl.BlockSpec((blk, H), lambda i: (i, 0)),
                   pl.BlockSpec((blk, H), lambda i: (i, 0))],
        out_shape=[jax.ShapeDtypeStruct((M, H), _F32),
                   jax.ShapeDtypeStruct((M, H), _F32)],
    )(x, w0a, b0.reshape(1, -1), w0b)


def _edge_pass(xd, xs, e, w0c, w1, b1, w2, b2, g, be, blk):
    M, H = e.shape
    grid = M // blk

    def body(xd_ref, xs_ref, e_ref, w0c_ref, w1_ref, b1_ref, w2_ref, b2_ref,
             g_ref, be_ref, o_ref):
        ev = e_ref[...]
        h = jnp.maximum(xd_ref[...] + xs_ref[...] + _dot(ev, w0c_ref[...]), 0.0)
        h = jnp.maximum(_dot(h, w1_ref[...]) + b1_ref[...], 0.0)
        u = _dot(h, w2_ref[...]) + b2_ref[...]
        o_ref[...] = _ln(u, g_ref[...], be_ref[...]) + ev

    full = lambda s: pl.BlockSpec(s, lambda i: (0, 0))
    row = pl.BlockSpec((blk, H), lambda i: (i, 0))
    return pl.pallas_call(
        body,
        grid=(grid,),
        in_specs=[row, row, row, full((H, H)), full((H, H)), full((1, H)),
                  full((H, H)), full((1, H)), full((1, H)), full((1, H))],
        out_specs=row,
        out_shape=jax.ShapeDtypeStruct((M, H), _F32),
    )(xd, xs, e, w0c, w1, b1.reshape(1, -1), w2, b2.reshape(1, -1),
      g.reshape(1, -1), be.reshape(1, -1))


def _node_pass(x, p0, p1, c0, c1, v0a, v0b, cb0, v1, cb1, v2, cb2, g, be, blk):
    M, H = x.shape
    grid = M // blk
    CW = c0.shape[1]

    def body(x_ref, p0_ref, p1_ref, c0_ref, c1_ref, va_ref, vb_ref, b0_ref,
             v1_ref, b1_ref, v2_ref, b2_ref, g_ref, be_ref, o_ref):
        xv = x_ref[...]
        aggr = p0_ref[...] + p1_ref[...]
        h = jnp.maximum(_dot(xv, va_ref[...]) + _dot(aggr, vb_ref[...])
                        + b0_ref[...], 0.0)
        h = jnp.maximum(_dot(h, v1_ref[...]) + b1_ref[...], 0.0)
        u = _dot(h, v2_ref[...]) + b2_ref[...]
        xu = _ln(u, g_ref[...], be_ref[...]) + xv
        cnt = jnp.sum(c0_ref[...] + c1_ref[...], axis=1, keepdims=True)
        o_ref[...] = jnp.where(cnt > 0.0, xu, xv)

    full = lambda s: pl.BlockSpec(s, lambda i: (0, 0))
    row = pl.BlockSpec((blk, H), lambda i: (i, 0))
    crow = pl.BlockSpec((blk, CW), lambda i: (i, 0))
    return pl.pallas_call(
        body,
        grid=(grid,),
        in_specs=[row, row, row, crow, crow, full((H, H)), full((H, H)),
                  full((1, H)), full((H, H)), full((1, H)), full((H, H)),
                  full((1, H)), full((1, H)), full((1, H))],
        out_specs=row,
        out_shape=jax.ShapeDtypeStruct((M, H), _F32),
    )(x, p0, p1, c0, c1, v0a, v0b, cb0.reshape(1, -1), v1, cb1.reshape(1, -1),
      v2, cb2.reshape(1, -1), g.reshape(1, -1), be.reshape(1, -1))


def _make_gather2(NT, D, E_pad, NC, NS):
    NW = NC * NS
    EPW = E_pad // NW
    CH = EPW // 128
    mesh = plsc.VectorSubcoreMesh(core_axis_name="c", subcore_axis_name="s")

    @functools.partial(
        pl.kernel, mesh=mesh,
        out_type=[jax.ShapeDtypeStruct((E_pad, D), _F32),
                  jax.ShapeDtypeStruct((E_pad, D), _F32)],
        scratch_types=[pltpu.VMEM((CH, 128), jnp.int32),
                       pltpu.VMEM((CH, 128), jnp.int32),
                       pltpu.VMEM((128, D), _F32),
                       pltpu.VMEM((128, D), _F32),
                       pltpu.VMEM((128, D), _F32),
                       pltpu.VMEM((128, D), _F32),
                       pltpu.SemaphoreType.DMA,
                       pltpu.SemaphoreType.DMA,
                       pltpu.SemaphoreType.DMA,
                       pltpu.SemaphoreType.DMA,
                       pltpu.SemaphoreType.DMA,
                       pltpu.SemaphoreType.DMA,
                       pltpu.SemaphoreType.DMA,
                       pltpu.SemaphoreType.DMA])
    def gather2(p_hbm, q_hbm, dst_hbm, src_hbm, xd_out, xs_out,
                dv, sv, ra0, rb0, ra1, rb1,
                sa0, sb0, sa1, sb1, swa0, swb0, swa1, swb1):
        c = lax.axis_index("c")
        s = lax.axis_index("s")
        wid = s * NC + c
        rbase = wid * CH
        ebase = wid * EPW
        pltpu.sync_copy(dst_hbm.at[pl.ds(rbase, CH)], dv)
        pltpu.sync_copy(src_hbm.at[pl.ds(rbase, CH)], sv)

        def body(i, carry):
            j0 = 2 * i
            j1 = j0 + 1
            a0 = pltpu.async_copy(p_hbm.at[dv.at[j0]], ra0, sa0)
            b0 = pltpu.async_copy(q_hbm.at[sv.at[j0]], rb0, sb0)
            a1 = pltpu.async_copy(p_hbm.at[dv.at[j1]], ra1, sa1)
            b1 = pltpu.async_copy(q_hbm.at[sv.at[j1]], rb1, sb1)
            a0.wait()
            wa0 = pltpu.async_copy(
                ra0, xd_out.at[pl.ds(ebase + j0 * 128, 128)], swa0)
            b0.wait()
            wb0 = pltpu.async_copy(
                rb0, xs_out.at[pl.ds(ebase + j0 * 128, 128)], swb0)
            a1.wait()
            wa1 = pltpu.async_copy(
                ra1, xd_out.at[pl.ds(ebase + j1 * 128, 128)], swa1)
            b1.wait()
            wb1 = pltpu.async_copy(
                rb1, xs_out.at[pl.ds(ebase + j1 * 128, 128)], swb1)
            wa0.wait()
            wb0.wait()
            wa1.wait()
            wb1.wait()
            return carry

        lax.fori_loop(0, CH // 2, body, 0)

    return gather2


def _make_scatter(NP, D, CW, E_pad, NC, NS):
    NW = NC * NS
    EPW = E_pad // NW
    CH = EPW // 128
    NPT = NP // NS
    mesh = plsc.VectorSubcoreMesh(core_axis_name="c", subcore_axis_name="s")

    _chunks = []
    _off = 0
    while _off < NPT:
        _w = min(128, NPT - _off)
        _chunks.append((_off, _w))
        _off += _w

    @functools.partial(
        pl.kernel, mesh=mesh,
        out_type=jax.ShapeDtypeStruct((NC * NP, D), _F32),
        scratch_types=[pltpu.VMEM((CH, 128), jnp.int32),
                       pltpu.VMEM((128, D), _F32),
                       pltpu.VMEM((128, D), _F32),
                       pltpu.VMEM_SHARED((NP, D), _F32),
                       pltpu.SemaphoreType.DMA,
                       pltpu.SemaphoreType.DMA])
    def scatter(vals_hbm, dst_hbm, zd_hbm, part_out,
                dv, vv, vv1, aggr_sh, s0, s1):
        c = lax.axis_index("c")
        s = lax.axis_index("s")
        wid = s * NC + c
        rb = s * NPT
        ob = c * NP
        pltpu.sync_copy(zd_hbm.at[pl.ds(0, 128)], vv)
        for off, w in _chunks:
            pltpu.sync_copy(vv.at[pl.ds(0, w)],
                            aggr_sh.at[pl.ds(rb + off, w)])
        pltpu.sync_copy(dst_hbm.at[pl.ds(wid * CH, CH)], dv)
        plsc.subcore_barrier()

        def body(i, carry):
            j0 = 2 * i
            j1 = j0 + 1
            c0 = pltpu.async_copy(
                vals_hbm.at[pl.ds(wid * EPW + j0 * 128, 128)], vv, s0)
            c1 = pltpu.async_copy(
                vals_hbm.at[pl.ds(wid * EPW + j1 * 128, 128)], vv1, s1)
            c0.wait()
            pltpu.sync_copy(vv, aggr_sh.at[dv.at[j0]], add=True)
            c1.wait()
            pltpu.sync_copy(vv1, aggr_sh.at[dv.at[j1]], add=True)
            return carry

        lax.fori_loop(0, CH // 2, body, 0)
        plsc.subcore_barrier()
        for off, w in _chunks:
            pltpu.sync_copy(aggr_sh.at[pl.ds(rb + off, w)],
                            vv.at[pl.ds(0, w)])
            pltpu.sync_copy(vv.at[pl.ds(0, w)],
                            part_out.at[pl.ds(ob + rb + off, w)])

    return scatter


def _make_count(NP, D, E_pad, NC, NS):
    NW = NC * NS
    EPW = E_pad // NW
    CH = EPW // 128
    NPT = NP // NS
    mesh = plsc.VectorSubcoreMesh(core_axis_name="c", subcore_axis_name="s")

    _chunks = []
    _off = 0
    while _off < NPT:
        _w = min(128, NPT - _off)
        _chunks.append((_off, _w))
        _off += _w

    @functools.partial(
        pl.kernel, mesh=mesh,
        out_type=jax.ShapeDtypeStruct((NC * NP, D), _F32),
        scratch_types=[pltpu.VMEM((1, 128), jnp.int32),
                       pltpu.VMEM((128, D), _F32),
                       pltpu.VMEM_SHARED((NP, D), _F32)])
    def count(dst_hbm, zd_hbm, ones_hbm, cnt_out, dv, vv, cnt_sh):
        c = lax.axis_index("c")
        s = lax.axis_index("s")
        wid = s * NC + c
        rb = s * NPT
        ob = c * NP
        pltpu.sync_copy(zd_hbm.at[pl.ds(0, 128)], vv)
        for off, w in _chunks:
            pltpu.sync_copy(vv.at[pl.ds(0, w)],
                            cnt_sh.at[pl.ds(rb + off, w)])
        pltpu.sync_copy(ones_hbm, vv)
        plsc.subcore_barrier()

        def body(j, carry):
            pltpu.sync_copy(dst_hbm.at[pl.ds(wid * CH + j, 1)], dv)
            pltpu.sync_copy(vv, cnt_sh.at[dv.at[0]], add=True)
            return carry

        lax.fori_loop(0, CH, body, 0)
        plsc.subcore_barrier()
        for off, w in _chunks:
            pltpu.sync_copy(cnt_sh.at[pl.ds(rb + off, w)],
                            vv.at[pl.ds(0, w)])
            pltpu.sync_copy(vv.at[pl.ds(0, w)],
                            cnt_out.at[pl.ds(ob + rb + off, w)])

    return count


def kernel(x, edge_index, edge_attr, params):
    N, D = x.shape
    E = edge_attr.shape[0]
    H = params["enc_node"]["Ws"][2].shape[1]
    CW = 16

    info = plsc.get_sparse_core_info()
    NC, NS = info.num_cores, info.num_subcores
    NW = NC * NS
    E_pad = ((E + NW * 256 - 1) // (NW * 256)) * (NW * 256)
    NP = ((N + 1 + 8 * NS - 1) // (8 * NS)) * (8 * NS)

    src = edge_index[0].astype(jnp.int32)
    dst = edge_index[1].astype(jnp.int32)
    pad = E_pad - E
    dst_g = jnp.concatenate([dst, jnp.zeros((pad,), jnp.int32)]).reshape(-1, 128)
    src_g = jnp.concatenate([src, jnp.zeros((pad,), jnp.int32)]).reshape(-1, 128)
    dst_s = jnp.concatenate([dst, jnp.full((pad,), N, jnp.int32)]).reshape(-1, 128)
    ea_pad = jnp.concatenate(
        [edge_attr, jnp.zeros((pad, edge_attr.shape[1]), _F32)])
    zd = jnp.zeros((NP, H), _F32)
    ones_h = jnp.ones((128, H), _F32)

    gather2 = _make_gather2(N, H, E_pad, NC, NS)
    scatter = _make_scatter(NP, H, CW, E_pad, NC, NS)
    count = _make_count(NP, H, E_pad, NC, NS)

    NBLK = 1000
    EBLK = 1024

    enc_n = params["enc_node"]
    xh = _mlp3(x, enc_n["Ws"], enc_n["bs"], enc_n["ln"], NBLK)
    enc_e = params["enc_edge"]
    e = _mlp3(ea_pad, enc_e["Ws"], enc_e["bs"], enc_e["ln"], EBLK)

    cnts = count(dst_s, zd, ones_h).reshape(NC, NP, H)
    c0, c1 = cnts[0, :N, :CW], cnts[1, :N, :CW]

    for layer in params["proc"]:
        em = layer["edge_mlp"]
        W0 = em["Ws"][0]
        P, Q = _pq(xh, W0[:H], W0[H:2 * H], em["bs"][0], NBLK)
        xd, xs = gather2(P, Q, dst_g, src_g)
        e = _edge_pass(xd, xs, e, W0[2 * H:], em["Ws"][1], em["bs"][1],
                       em["Ws"][2], em["bs"][2], em["ln"][0], em["ln"][1], EBLK)
        parts = scatter(e, dst_s, zd).reshape(NC, NP, H)
        nm = layer["node_mlp"]
        V0 = nm["Ws"][0]
        xh = _node_pass(xh, parts[0, :N], parts[1, :N], c0, c1,
                        V0[:H], V0[H:], nm["bs"][0], nm["Ws"][1],
                        nm["bs"][1], nm["Ws"][2], nm["bs"][2], nm["ln"][0],
                        nm["ln"][1], NBLK)

    dn = params["dec_node"]
    on = dn["Ws"][2].shape[1]
    w3n = jnp.pad(dn["Ws"][2], ((0, 0), (0, 8 - on)))
    b3n = jnp.pad(dn["bs"][2], (0, 8 - on))
    node_out = _mlp3(x=xh, Ws=[dn["Ws"][0], dn["Ws"][1], w3n],
                     bs=[dn["bs"][0], dn["bs"][1], b3n], ln=None, blk=NBLK)
    de = params["dec_edge"]
    oe = de["Ws"][2].shape[1]
    w3e = jnp.pad(de["Ws"][2], ((0, 0), (0, 8 - oe)))
    b3e = jnp.pad(de["bs"][2], (0, 8 - oe))
    edge_out = _mlp3(x=e, Ws=[de["Ws"][0], de["Ws"][1], w3e],
                     bs=[de["bs"][0], de["bs"][1], b3e], ln=None, blk=EBLK)
    return (node_out[:, :on], edge_out[:E, :oe])


# speedup vs baseline: 1.2277x; 1.0379x over previous
import functools

import jax
import jax.numpy as jnp
from jax import lax
from jax.experimental import pallas as pl
from jax.experimental.pallas import tpu as pltpu
from jax.experimental.pallas import tpu_sc as plsc

_F32 = jnp.float32


def _ln(u, g, b):
    mu = jnp.mean(u, axis=-1, keepdims=True)
    var = jnp.mean((u - mu) ** 2, axis=-1, keepdims=True)
    return (u - mu) * lax.rsqrt(var + 1e-5) * g + b


def _dot(a, w):
    return jnp.dot(a, w, preferred_element_type=_F32)


def _mlp3(x, Ws, bs, ln, blk):
    M, Din = x.shape
    Dout = Ws[2].shape[1]
    grid = M // blk
    has_ln = ln is not None

    def body(x_ref, w1_ref, b1_ref, w2_ref, b2_ref, w3_ref, b3_ref, *rest):
        if has_ln:
            g_ref, be_ref, o_ref = rest
        else:
            (o_ref,) = rest
        h = jnp.maximum(_dot(x_ref[...], w1_ref[...]) + b1_ref[...], 0.0)
        h = jnp.maximum(_dot(h, w2_ref[...]) + b2_ref[...], 0.0)
        u = _dot(h, w3_ref[...]) + b3_ref[...]
        if has_ln:
            u = _ln(u, g_ref[...], be_ref[...])
        o_ref[...] = u

    full = lambda s: pl.BlockSpec(s, lambda i: (0, 0))
    in_specs = [
        pl.BlockSpec((blk, Din), lambda i: (i, 0)),
        full(Ws[0].shape), full((1, Ws[0].shape[1])),
        full(Ws[1].shape), full((1, Ws[1].shape[1])),
        full(Ws[2].shape), full((1, Dout)),
    ]
    args = [x, Ws[0], bs[0].reshape(1, -1), Ws[1], bs[1].reshape(1, -1),
            Ws[2], bs[2].reshape(1, -1)]
    if has_ln:
        in_specs += [full((1, Dout)), full((1, Dout))]
        args += [ln[0].reshape(1, -1), ln[1].reshape(1, -1)]
    return pl.pallas_call(
        body,
        grid=(grid,),
        in_specs=in_specs,
        out_specs=pl.BlockSpec((blk, Dout), lambda i: (i, 0)),
        out_shape=jax.ShapeDtypeStruct((M, Dout), _F32),
    )(*args)


def _pq(x, w0a, w0b, b0, blk):
    M, D = x.shape
    H = w0a.shape[1]
    grid = M // blk

    def body(x_ref, wa_ref, b_ref, wb_ref, p_ref, q_ref):
        xv = x_ref[...]
        p_ref[...] = _dot(xv, wa_ref[...]) + b_ref[...]
        q_ref[...] = _dot(xv, wb_ref[...])

    full = lambda s: pl.BlockSpec(s, lambda i: (0, 0))
    return pl.pallas_call(
        body,
        grid=(grid,),
        in_specs=[pl.BlockSpec((blk, D), lambda i: (i, 0)),
                  full((D, H)), full((1, H)), full((D, H))],
        out_specs=[pl.BlockSpec((blk, H), lambda i: (i, 0)),
                   pl.BlockSpec((blk, H), lambda i: (i, 0))],
        out_shape=[jax.ShapeDtypeStruct((M, H), _F32),
                   jax.ShapeDtypeStruct((M, H), _F32)],
    )(x, w0a, b0.reshape(1, -1), w0b)


def _edge_pass(xd, xs, e, w0c, w1, b1, w2, b2, g, be, blk):
    M, H = e.shape
    grid = M // blk

    def body(xd_ref, xs_ref, e_ref, w0c_ref, w1_ref, b1_ref, w2_ref, b2_ref,
             g_ref, be_ref, o_ref):
        ev = e_ref[...]
        h = jnp.maximum(xd_ref[...] + xs_ref[...] + _dot(ev, w0c_ref[...]), 0.0)
        h = jnp.maximum(_dot(h, w1_ref[...]) + b1_ref[...], 0.0)
        u = _dot(h, w2_ref[...]) + b2_ref[...]
        o_ref[...] = _ln(u, g_ref[...], be_ref[...]) + ev

    full = lambda s: pl.BlockSpec(s, lambda i: (0, 0))
    row = pl.BlockSpec((blk, H), lambda i: (i, 0))
    return pl.pallas_call(
        body,
        grid=(grid,),
        in_specs=[row, row, row, full((H, H)), full((H, H)), full((1, H)),
                  full((H, H)), full((1, H)), full((1, H)), full((1, H))],
        out_specs=row,
        out_shape=jax.ShapeDtypeStruct((M, H), _F32),
    )(xd, xs, e, w0c, w1, b1.reshape(1, -1), w2, b2.reshape(1, -1),
      g.reshape(1, -1), be.reshape(1, -1))


def _node_pass(x, p0, p1, c0, c1, v0a, v0b, cb0, v1, cb1, v2, cb2, g, be, blk):
    M, H = x.shape
    grid = M // blk
    CW = c0.shape[1]

    def body(x_ref, p0_ref, p1_ref, c0_ref, c1_ref, va_ref, vb_ref, b0_ref,
             v1_ref, b1_ref, v2_ref, b2_ref, g_ref, be_ref, o_ref):
        xv = x_ref[...]
        aggr = p0_ref[...] + p1_ref[...]
        h = jnp.maximum(_dot(xv, va_ref[...]) + _dot(aggr, vb_ref[...])
                        + b0_ref[...], 0.0)
        h = jnp.maximum(_dot(h, v1_ref[...]) + b1_ref[...], 0.0)
        u = _dot(h, v2_ref[...]) + b2_ref[...]
        xu = _ln(u, g_ref[...], be_ref[...]) + xv
        cnt = jnp.sum(c0_ref[...] + c1_ref[...], axis=1, keepdims=True)
        o_ref[...] = jnp.where(cnt > 0.0, xu, xv)

    full = lambda s: pl.BlockSpec(s, lambda i: (0, 0))
    row = pl.BlockSpec((blk, H), lambda i: (i, 0))
    crow = pl.BlockSpec((blk, CW), lambda i: (i, 0))
    return pl.pallas_call(
        body,
        grid=(grid,),
        in_specs=[row, row, row, crow, crow, full((H, H)), full((H, H)),
                  full((1, H)), full((H, H)), full((1, H)), full((H, H)),
                  full((1, H)), full((1, H)), full((1, H))],
        out_specs=row,
        out_shape=jax.ShapeDtypeStruct((M, H), _F32),
    )(x, p0, p1, c0, c1, v0a, v0b, cb0.reshape(1, -1), v1, cb1.reshape(1, -1),
      v2, cb2.reshape(1, -1), g.reshape(1, -1), be.reshape(1, -1))


def _make_gather2(NT, D, E_pad, NC, NS):
    NW = NC * NS
    EPW = E_pad // NW
    CH = EPW // 128
    mesh = plsc.VectorSubcoreMesh(core_axis_name="c", subcore_axis_name="s")

    NBUF = 3
    LAG = 1

    @functools.partial(
        pl.kernel, mesh=mesh,
        out_type=[jax.ShapeDtypeStruct((E_pad, D), _F32),
                  jax.ShapeDtypeStruct((E_pad, D), _F32)],
        scratch_types=(
            [pltpu.VMEM((CH, 128), jnp.int32),
             pltpu.VMEM((CH, 128), jnp.int32)]
            + [pltpu.VMEM((128, D), _F32)] * (2 * NBUF)
            + [pltpu.SemaphoreType.DMA] * (4 * NBUF)))
    def gather2(p_hbm, q_hbm, dst_hbm, src_hbm, xd_out, xs_out,
                dv, sv, *rest):
        ras = rest[0:NBUF]
        rbs = rest[NBUF:2 * NBUF]
        gsa = rest[2 * NBUF:3 * NBUF]
        gsb = rest[3 * NBUF:4 * NBUF]
        wsa = rest[4 * NBUF:5 * NBUF]
        wsb = rest[5 * NBUF:6 * NBUF]
        c = lax.axis_index("c")
        s = lax.axis_index("s")
        wid = s * NC + c
        rbase = wid * CH
        ebase = wid * EPW
        pltpu.sync_copy(dst_hbm.at[pl.ds(rbase, CH)], dv)
        pltpu.sync_copy(src_hbm.at[pl.ds(rbase, CH)], sv)

        G = [None] * CH
        W = [None] * CH
        for t in range(CH + LAG):
            i = t
            if i < CH:
                b = i % NBUF
                if i >= NBUF:
                    W[i - NBUF][0].wait()
                    W[i - NBUF][1].wait()
                G[i] = (
                    pltpu.async_copy(p_hbm.at[dv.at[i]], ras[b], gsa[b]),
                    pltpu.async_copy(q_hbm.at[sv.at[i]], rbs[b], gsb[b]))
            j = t - LAG
            if 0 <= j < CH:
                b = j % NBUF
                G[j][0].wait()
                W[j] = (
                    pltpu.async_copy(
                        ras[b], xd_out.at[pl.ds(ebase + j * 128, 128)],
                        wsa[b]),)
                G[j][1].wait()
                W[j] = W[j] + (
                    pltpu.async_copy(
                        rbs[b], xs_out.at[pl.ds(ebase + j * 128, 128)],
                        wsb[b]),)
        for j in range(max(0, CH - NBUF), CH):
            W[j][0].wait()
            W[j][1].wait()

    return gather2


def _make_scatter(NP, D, CW, E_pad, NC, NS):
    NW = NC * NS
    EPW = E_pad // NW
    CH = EPW // 128
    NPT = NP // NS
    mesh = plsc.VectorSubcoreMesh(core_axis_name="c", subcore_axis_name="s")

    _chunks = []
    _off = 0
    while _off < NPT:
        _w = min(128, NPT - _off)
        _chunks.append((_off, _w))
        _off += _w

    @functools.partial(
        pl.kernel, mesh=mesh,
        out_type=jax.ShapeDtypeStruct((NC * NP, D), _F32),
        scratch_types=[pltpu.VMEM((CH, 128), jnp.int32),
                       pltpu.VMEM((128, D), _F32),
                       pltpu.VMEM((128, D), _F32),
                       pltpu.VMEM_SHARED((NP, D), _F32),
                       pltpu.SemaphoreType.DMA,
                       pltpu.SemaphoreType.DMA])
    def scatter(vals_hbm, dst_hbm, zd_hbm, part_out,
                dv, vv, vv1, aggr_sh, s0, s1):
        c = lax.axis_index("c")
        s = lax.axis_index("s")
        wid = s * NC + c
        rb = s * NPT
        ob = c * NP
        pltpu.sync_copy(zd_hbm.at[pl.ds(0, 128)], vv)
        for off, w in _chunks:
            pltpu.sync_copy(vv.at[pl.ds(0, w)],
                            aggr_sh.at[pl.ds(rb + off, w)])
        pltpu.sync_copy(dst_hbm.at[pl.ds(wid * CH, CH)], dv)
        plsc.subcore_barrier()

        def body(i, carry):
            j0 = 2 * i
            j1 = j0 + 1
            c0 = pltpu.async_copy(
                vals_hbm.at[pl.ds(wid * EPW + j0 * 128, 128)], vv, s0)
            c1 = pltpu.async_copy(
                vals_hbm.at[pl.ds(wid * EPW + j1 * 128, 128)], vv1, s1)
            c0.wait()
            pltpu.sync_copy(vv, aggr_sh.at[dv.at[j0]], add=True)
            c1.wait()
            pltpu.sync_copy(vv1, aggr_sh.at[dv.at[j1]], add=True)
            return carry

        lax.fori_loop(0, CH // 2, body, 0)
        plsc.subcore_barrier()
        for off, w in _chunks:
            pltpu.sync_copy(aggr_sh.at[pl.ds(rb + off, w)],
                            vv.at[pl.ds(0, w)])
            pltpu.sync_copy(vv.at[pl.ds(0, w)],
                            part_out.at[pl.ds(ob + rb + off, w)])

    return scatter


def _make_count(NP, D, E_pad, NC, NS):
    NW = NC * NS
    EPW = E_pad // NW
    CH = EPW // 128
    NPT = NP // NS
    mesh = plsc.VectorSubcoreMesh(core_axis_name="c", subcore_axis_name="s")

    _chunks = []
    _off = 0
    while _off < NPT:
        _w = min(128, NPT - _off)
        _chunks.append((_off, _w))
        _off += _w

    @functools.partial(
        pl.kernel, mesh=mesh,
        out_type=jax.ShapeDtypeStruct((NC * NP, D), _F32),
        scratch_types=[pltpu.VMEM((1, 128), jnp.int32),
                       pltpu.VMEM((128, D), _F32),
                       pltpu.VMEM_SHARED((NP, D), _F32)])
    def count(dst_hbm, zd_hbm, ones_hbm, cnt_out, dv, vv, cnt_sh):
        c = lax.axis_index("c")
        s = lax.axis_index("s")
        wid = s * NC + c
        rb = s * NPT
        ob = c * NP
        pltpu.sync_copy(zd_hbm.at[pl.ds(0, 128)], vv)
        for off, w in _chunks:
            pltpu.sync_copy(vv.at[pl.ds(0, w)],
                            cnt_sh.at[pl.ds(rb + off, w)])
        pltpu.sync_copy(ones_hbm, vv)
        plsc.subcore_barrier()

        def body(j, carry):
            pltpu.sync_copy(dst_hbm.at[pl.ds(wid * CH + j, 1)], dv)
            pltpu.sync_copy(vv, cnt_sh.at[dv.at[0]], add=True)
            return carry

        lax.fori_loop(0, CH, body, 0)
        plsc.subcore_barrier()
        for off, w in _chunks:
            pltpu.sync_copy(cnt_sh.at[pl.ds(rb + off, w)],
                            vv.at[pl.ds(0, w)])
            pltpu.sync_copy(vv.at[pl.ds(0, w)],
                            cnt_out.at[pl.ds(ob + rb + off, w)])

    return count


def kernel(x, edge_index, edge_attr, params):
    N, D = x.shape
    E = edge_attr.shape[0]
    H = params["enc_node"]["Ws"][2].shape[1]
    CW = 16

    info = plsc.get_sparse_core_info()
    NC, NS = info.num_cores, info.num_subcores
    NW = NC * NS
    E_pad = ((E + NW * 256 - 1) // (NW * 256)) * (NW * 256)
    NP = ((N + 1 + 8 * NS - 1) // (8 * NS)) * (8 * NS)

    src = edge_index[0].astype(jnp.int32)
    dst = edge_index[1].astype(jnp.int32)
    pad = E_pad - E
    dst_g = jnp.concatenate([dst, jnp.zeros((pad,), jnp.int32)]).reshape(-1, 128)
    src_g = jnp.concatenate([src, jnp.zeros((pad,), jnp.int32)]).reshape(-1, 128)
    dst_s = jnp.concatenate([dst, jnp.full((pad,), N, jnp.int32)]).reshape(-1, 128)
    ea_pad = jnp.concatenate(
        [edge_attr, jnp.zeros((pad, edge_attr.shape[1]), _F32)])
    zd = jnp.zeros((NP, H), _F32)
    ones_h = jnp.ones((128, H), _F32)

    gather2 = _make_gather2(N, H, E_pad, NC, NS)
    scatter = _make_scatter(NP, H, CW, E_pad, NC, NS)
    count = _make_count(NP, H, E_pad, NC, NS)

    NBLK = 1000
    EBLK = 1024

    enc_n = params["enc_node"]
    xh = _mlp3(x, enc_n["Ws"], enc_n["bs"], enc_n["ln"], NBLK)
    enc_e = params["enc_edge"]
    e = _mlp3(ea_pad, enc_e["Ws"], enc_e["bs"], enc_e["ln"], EBLK)

    cnts = count(dst_s, zd, ones_h).reshape(NC, NP, H)
    c0, c1 = cnts[0, :N, :CW], cnts[1, :N, :CW]

    for layer in params["proc"]:
        em = layer["edge_mlp"]
        W0 = em["Ws"][0]
        P, Q = _pq(xh, W0[:H], W0[H:2 * H], em["bs"][0], NBLK)
        xd, xs = gather2(P, Q, dst_g, src_g)
        e = _edge_pass(xd, xs, e, W0[2 * H:], em["Ws"][1], em["bs"][1],
                       em["Ws"][2], em["bs"][2], em["ln"][0], em["ln"][1], EBLK)
        parts = scatter(e, dst_s, zd).reshape(NC, NP, H)
        nm = layer["node_mlp"]
        V0 = nm["Ws"][0]
        xh = _node_pass(xh, parts[0, :N], parts[1, :N], c0, c1,
                        V0[:H], V0[H:], nm["bs"][0], nm["Ws"][1],
                        nm["bs"][1], nm["Ws"][2], nm["bs"][2], nm["ln"][0],
                        nm["ln"][1], NBLK)

    dn = params["dec_node"]
    on = dn["Ws"][2].shape[1]
    w3n = jnp.pad(dn["Ws"][2], ((0, 0), (0, 8 - on)))
    b3n = jnp.pad(dn["bs"][2], (0, 8 - on))
    node_out = _mlp3(x=xh, Ws=[dn["Ws"][0], dn["Ws"][1], w3n],
                     bs=[dn["bs"][0], dn["bs"][1], b3n], ln=None, blk=NBLK)
    de = params["dec_edge"]
    oe = de["Ws"][2].shape[1]
    w3e = jnp.pad(de["Ws"][2], ((0, 0), (0, 8 - oe)))
    b3e = jnp.pad(de["bs"][2], (0, 8 - oe))
    edge_out = _mlp3(x=e, Ws=[de["Ws"][0], de["Ws"][1], w3e],
                     bs=[de["bs"][0], de["bs"][1], b3e], ln=None, blk=EBLK)
    return (node_out[:, :on], edge_out[:E, :oe])


# speedup vs baseline: 1.2416x; 1.0113x over previous
import functools

import jax
import jax.numpy as jnp
from jax import lax
from jax.experimental import pallas as pl
from jax.experimental.pallas import tpu as pltpu
from jax.experimental.pallas import tpu_sc as plsc

_F32 = jnp.float32


def _ln(u, g, b):
    mu = jnp.mean(u, axis=-1, keepdims=True)
    var = jnp.mean((u - mu) ** 2, axis=-1, keepdims=True)
    return (u - mu) * lax.rsqrt(var + 1e-5) * g + b


def _dot(a, w):
    return jnp.dot(a, w, preferred_element_type=_F32)


def _mlp3(x, Ws, bs, ln, blk):
    M, Din = x.shape
    Dout = Ws[2].shape[1]
    grid = M // blk
    has_ln = ln is not None

    def body(x_ref, w1_ref, b1_ref, w2_ref, b2_ref, w3_ref, b3_ref, *rest):
        if has_ln:
            g_ref, be_ref, o_ref = rest
        else:
            (o_ref,) = rest
        h = jnp.maximum(_dot(x_ref[...], w1_ref[...]) + b1_ref[...], 0.0)
        h = jnp.maximum(_dot(h, w2_ref[...]) + b2_ref[...], 0.0)
        u = _dot(h, w3_ref[...]) + b3_ref[...]
        if has_ln:
            u = _ln(u, g_ref[...], be_ref[...])
        o_ref[...] = u

    full = lambda s: pl.BlockSpec(s, lambda i: (0, 0))
    in_specs = [
        pl.BlockSpec((blk, Din), lambda i: (i, 0)),
        full(Ws[0].shape), full((1, Ws[0].shape[1])),
        full(Ws[1].shape), full((1, Ws[1].shape[1])),
        full(Ws[2].shape), full((1, Dout)),
    ]
    args = [x, Ws[0], bs[0].reshape(1, -1), Ws[1], bs[1].reshape(1, -1),
            Ws[2], bs[2].reshape(1, -1)]
    if has_ln:
        in_specs += [full((1, Dout)), full((1, Dout))]
        args += [ln[0].reshape(1, -1), ln[1].reshape(1, -1)]
    return pl.pallas_call(
        body,
        grid=(grid,),
        in_specs=in_specs,
        out_specs=pl.BlockSpec((blk, Dout), lambda i: (i, 0)),
        out_shape=jax.ShapeDtypeStruct((M, Dout), _F32),
    )(*args)


def _pq(x, w0a, w0b, b0, blk):
    M, D = x.shape
    H = w0a.shape[1]
    grid = M // blk

    def body(x_ref, wa_ref, b_ref, wb_ref, p_ref, q_ref):
        xv = x_ref[...]
        p_ref[...] = _dot(xv, wa_ref[...]) + b_ref[...]
        q_ref[...] = _dot(xv, wb_ref[...])

    full = lambda s: pl.BlockSpec(s, lambda i: (0, 0))
    return pl.pallas_call(
        body,
        grid=(grid,),
        in_specs=[pl.BlockSpec((blk, D), lambda i: (i, 0)),
                  full((D, H)), full((1, H)), full((D, H))],
        out_specs=[pl.BlockSpec((blk, H), lambda i: (i, 0)),
                   pl.BlockSpec((blk, H), lambda i: (i, 0))],
        out_shape=[jax.ShapeDtypeStruct((M, H), _F32),
                   jax.ShapeDtypeStruct((M, H), _F32)],
    )(x, w0a, b0.reshape(1, -1), w0b)


def _edge_pass(xd, xs, e, w0c, w1, b1, w2, b2, g, be, blk):
    M, H = e.shape
    grid = M // blk

    def body(xd_ref, xs_ref, e_ref, w0c_ref, w1_ref, b1_ref, w2_ref, b2_ref,
             g_ref, be_ref, o_ref):
        ev = e_ref[...]
        h = jnp.maximum(xd_ref[...] + xs_ref[...] + _dot(ev, w0c_ref[...]), 0.0)
        h = jnp.maximum(_dot(h, w1_ref[...]) + b1_ref[...], 0.0)
        u = _dot(h, w2_ref[...]) + b2_ref[...]
        o_ref[...] = _ln(u, g_ref[...], be_ref[...]) + ev

    full = lambda s: pl.BlockSpec(s, lambda i: (0, 0))
    row = pl.BlockSpec((blk, H), lambda i: (i, 0))
    return pl.pallas_call(
        body,
        grid=(grid,),
        in_specs=[row, row, row, full((H, H)), full((H, H)), full((1, H)),
                  full((H, H)), full((1, H)), full((1, H)), full((1, H))],
        out_specs=row,
        out_shape=jax.ShapeDtypeStruct((M, H), _F32),
    )(xd, xs, e, w0c, w1, b1.reshape(1, -1), w2, b2.reshape(1, -1),
      g.reshape(1, -1), be.reshape(1, -1))


def _node_pass(x, p0, p1, c0, c1, v0a, v0b, cb0, v1, cb1, v2, cb2, g, be, blk):
    M, H = x.shape
    grid = M // blk
    CW = c0.shape[1]

    def body(x_ref, p0_ref, p1_ref, c0_ref, c1_ref, va_ref, vb_ref, b0_ref,
             v1_ref, b1_ref, v2_ref, b2_ref, g_ref, be_ref, o_ref):
        xv = x_ref[...]
        aggr = p0_ref[...] + p1_ref[...]
        h = jnp.maximum(_dot(xv, va_ref[...]) + _dot(aggr, vb_ref[...])
                        + b0_ref[...], 0.0)
        h = jnp.maximum(_dot(h, v1_ref[...]) + b1_ref[...], 0.0)
        u = _dot(h, v2_ref[...]) + b2_ref[...]
        xu = _ln(u, g_ref[...], be_ref[...]) + xv
        cnt = jnp.sum(c0_ref[...] + c1_ref[...], axis=1, keepdims=True)
        o_ref[...] = jnp.where(cnt > 0.0, xu, xv)

    full = lambda s: pl.BlockSpec(s, lambda i: (0, 0))
    row = pl.BlockSpec((blk, H), lambda i: (i, 0))
    crow = pl.BlockSpec((blk, CW), lambda i: (i, 0))
    return pl.pallas_call(
        body,
        grid=(grid,),
        in_specs=[row, row, row, crow, crow, full((H, H)), full((H, H)),
                  full((1, H)), full((H, H)), full((1, H)), full((H, H)),
                  full((1, H)), full((1, H)), full((1, H))],
        out_specs=row,
        out_shape=jax.ShapeDtypeStruct((M, H), _F32),
    )(x, p0, p1, c0, c1, v0a, v0b, cb0.reshape(1, -1), v1, cb1.reshape(1, -1),
      v2, cb2.reshape(1, -1), g.reshape(1, -1), be.reshape(1, -1))


def _make_gather2(NT, D, E_pad, NC, NS):
    NW = NC * NS
    EPW = E_pad // NW
    CH = EPW // 128
    mesh = plsc.VectorSubcoreMesh(core_axis_name="c", subcore_axis_name="s")

    NBUF = 3
    LAG = 1

    @functools.partial(
        pl.kernel, mesh=mesh,
        out_type=[jax.ShapeDtypeStruct((E_pad, D), _F32),
                  jax.ShapeDtypeStruct((E_pad, D), _F32)],
        scratch_types=(
            [pltpu.VMEM((CH, 128), jnp.int32),
             pltpu.VMEM((CH, 128), jnp.int32)]
            + [pltpu.VMEM((128, D), _F32)] * (2 * NBUF)
            + [pltpu.SemaphoreType.DMA] * (4 * NBUF)))
    def gather2(p_hbm, q_hbm, dst_hbm, src_hbm, xd_out, xs_out,
                dv, sv, *rest):
        ras = rest[0:NBUF]
        rbs = rest[NBUF:2 * NBUF]
        gsa = rest[2 * NBUF:3 * NBUF]
        gsb = rest[3 * NBUF:4 * NBUF]
        wsa = rest[4 * NBUF:5 * NBUF]
        wsb = rest[5 * NBUF:6 * NBUF]
        c = lax.axis_index("c")
        s = lax.axis_index("s")
        wid = s * NC + c
        rbase = wid * CH
        ebase = wid * EPW
        pltpu.sync_copy(dst_hbm.at[pl.ds(rbase, CH)], dv)
        pltpu.sync_copy(src_hbm.at[pl.ds(rbase, CH)], sv)

        G = [None] * CH
        W = [None] * CH
        for t in range(CH + LAG):
            i = t
            if i < CH:
                b = i % NBUF
                if i >= NBUF:
                    W[i - NBUF][0].wait()
                    W[i - NBUF][1].wait()
                G[i] = (
                    pltpu.async_copy(p_hbm.at[dv.at[i]], ras[b], gsa[b]),
                    pltpu.async_copy(q_hbm.at[sv.at[i]], rbs[b], gsb[b]))
            j = t - LAG
            if 0 <= j < CH:
                b = j % NBUF
                G[j][0].wait()
                W[j] = (
                    pltpu.async_copy(
                        ras[b], xd_out.at[pl.ds(ebase + j * 128, 128)],
                        wsa[b]),)
                G[j][1].wait()
                W[j] = W[j] + (
                    pltpu.async_copy(
                        rbs[b], xs_out.at[pl.ds(ebase + j * 128, 128)],
                        wsb[b]),)
        for j in range(max(0, CH - NBUF), CH):
            W[j][0].wait()
            W[j][1].wait()

    return gather2


def _make_scatter(NP, D, CW, E_pad, NC, NS):
    NW = NC * NS
    EPW = E_pad // NW
    CH = EPW // 128
    NPT = NP // NS
    mesh = plsc.VectorSubcoreMesh(core_axis_name="c", subcore_axis_name="s")

    _chunks = []
    _off = 0
    while _off < NPT:
        _w = min(128, NPT - _off)
        _chunks.append((_off, _w))
        _off += _w

    @functools.partial(
        pl.kernel, mesh=mesh,
        out_type=jax.ShapeDtypeStruct((NC * NP, D), _F32),
        scratch_types=[pltpu.VMEM((CH, 128), jnp.int32),
                       pltpu.VMEM((128, D), _F32),
                       pltpu.VMEM((128, D), _F32),
                       pltpu.VMEM_SHARED((NP, D), _F32),
                       pltpu.SemaphoreType.DMA,
                       pltpu.SemaphoreType.DMA])
    def scatter(vals_hbm, dst_hbm, zd_hbm, part_out,
                dv, vv, vv1, aggr_sh, s0, s1):
        c = lax.axis_index("c")
        s = lax.axis_index("s")
        wid = s * NC + c
        rb = s * NPT
        ob = c * NP
        pltpu.sync_copy(zd_hbm.at[pl.ds(0, 128)], vv)
        for off, w in _chunks:
            pltpu.sync_copy(vv.at[pl.ds(0, w)],
                            aggr_sh.at[pl.ds(rb + off, w)])
        pltpu.sync_copy(dst_hbm.at[pl.ds(wid * CH, CH)], dv)
        plsc.subcore_barrier()

        bufs = (vv, vv1)
        sems = (s0, s1)
        L = [None] * CH
        for i in range(CH):
            L[i] = pltpu.async_copy(
                vals_hbm.at[pl.ds(wid * EPW + i * 128, 128)],
                bufs[i % 2], sems[i % 2])
            if i >= 1:
                j = i - 1
                L[j].wait()
                pltpu.sync_copy(bufs[j % 2], aggr_sh.at[dv.at[j]], add=True)
        L[CH - 1].wait()
        pltpu.sync_copy(bufs[(CH - 1) % 2], aggr_sh.at[dv.at[CH - 1]],
                        add=True)
        plsc.subcore_barrier()
        for off, w in _chunks:
            pltpu.sync_copy(aggr_sh.at[pl.ds(rb + off, w)],
                            vv.at[pl.ds(0, w)])
            pltpu.sync_copy(vv.at[pl.ds(0, w)],
                            part_out.at[pl.ds(ob + rb + off, w)])

    return scatter


def _make_count(NP, D, E_pad, NC, NS):
    NW = NC * NS
    EPW = E_pad // NW
    CH = EPW // 128
    NPT = NP // NS
    mesh = plsc.VectorSubcoreMesh(core_axis_name="c", subcore_axis_name="s")

    _chunks = []
    _off = 0
    while _off < NPT:
        _w = min(128, NPT - _off)
        _chunks.append((_off, _w))
        _off += _w

    @functools.partial(
        pl.kernel, mesh=mesh,
        out_type=jax.ShapeDtypeStruct((NC * NP, D), _F32),
        scratch_types=[pltpu.VMEM((1, 128), jnp.int32),
                       pltpu.VMEM((128, D), _F32),
                       pltpu.VMEM_SHARED((NP, D), _F32)])
    def count(dst_hbm, zd_hbm, ones_hbm, cnt_out, dv, vv, cnt_sh):
        c = lax.axis_index("c")
        s = lax.axis_index("s")
        wid = s * NC + c
        rb = s * NPT
        ob = c * NP
        pltpu.sync_copy(zd_hbm.at[pl.ds(0, 128)], vv)
        for off, w in _chunks:
            pltpu.sync_copy(vv.at[pl.ds(0, w)],
                            cnt_sh.at[pl.ds(rb + off, w)])
        pltpu.sync_copy(ones_hbm, vv)
        plsc.subcore_barrier()

        def body(j, carry):
            pltpu.sync_copy(dst_hbm.at[pl.ds(wid * CH + j, 1)], dv)
            pltpu.sync_copy(vv, cnt_sh.at[dv.at[0]], add=True)
            return carry

        lax.fori_loop(0, CH, body, 0)
        plsc.subcore_barrier()
        for off, w in _chunks:
            pltpu.sync_copy(cnt_sh.at[pl.ds(rb + off, w)],
                            vv.at[pl.ds(0, w)])
            pltpu.sync_copy(vv.at[pl.ds(0, w)],
                            cnt_out.at[pl.ds(ob + rb + off, w)])

    return count


def kernel(x, edge_index, edge_attr, params):
    N, D = x.shape
    E = edge_attr.shape[0]
    H = params["enc_node"]["Ws"][2].shape[1]
    CW = 16

    info = plsc.get_sparse_core_info()
    NC, NS = info.num_cores, info.num_subcores
    NW = NC * NS
    E_pad = ((E + NW * 256 - 1) // (NW * 256)) * (NW * 256)
    NP = ((N + 1 + 8 * NS - 1) // (8 * NS)) * (8 * NS)

    src = edge_index[0].astype(jnp.int32)
    dst = edge_index[1].astype(jnp.int32)
    pad = E_pad - E
    dst_g = jnp.concatenate([dst, jnp.zeros((pad,), jnp.int32)]).reshape(-1, 128)
    src_g = jnp.concatenate([src, jnp.zeros((pad,), jnp.int32)]).reshape(-1, 128)
    dst_s = jnp.concatenate([dst, jnp.full((pad,), N, jnp.int32)]).reshape(-1, 128)
    ea_pad = jnp.concatenate(
        [edge_attr, jnp.zeros((pad, edge_attr.shape[1]), _F32)])
    zd = jnp.zeros((NP, H), _F32)
    ones_h = jnp.ones((128, H), _F32)

    gather2 = _make_gather2(N, H, E_pad, NC, NS)
    scatter = _make_scatter(NP, H, CW, E_pad, NC, NS)
    count = _make_count(NP, H, E_pad, NC, NS)

    NBLK = 1000
    EBLK = 1024

    enc_n = params["enc_node"]
    xh = _mlp3(x, enc_n["Ws"], enc_n["bs"], enc_n["ln"], NBLK)
    enc_e = params["enc_edge"]
    e = _mlp3(ea_pad, enc_e["Ws"], enc_e["bs"], enc_e["ln"], EBLK)

    cnts = count(dst_s, zd, ones_h).reshape(NC, NP, H)
    c0, c1 = cnts[0, :N, :CW], cnts[1, :N, :CW]

    for layer in params["proc"]:
        em = layer["edge_mlp"]
        W0 = em["Ws"][0]
        P, Q = _pq(xh, W0[:H], W0[H:2 * H], em["bs"][0], NBLK)
        xd, xs = gather2(P, Q, dst_g, src_g)
        e = _edge_pass(xd, xs, e, W0[2 * H:], em["Ws"][1], em["bs"][1],
                       em["Ws"][2], em["bs"][2], em["ln"][0], em["ln"][1], EBLK)
        parts = scatter(e, dst_s, zd).reshape(NC, NP, H)
        nm = layer["node_mlp"]
        V0 = nm["Ws"][0]
        xh = _node_pass(xh, parts[0, :N], parts[1, :N], c0, c1,
                        V0[:H], V0[H:], nm["bs"][0], nm["Ws"][1],
                        nm["bs"][1], nm["Ws"][2], nm["bs"][2], nm["ln"][0],
                        nm["ln"][1], NBLK)

    dn = params["dec_node"]
    on = dn["Ws"][2].shape[1]
    w3n = jnp.pad(dn["Ws"][2], ((0, 0), (0, 8 - on)))
    b3n = jnp.pad(dn["bs"][2], (0, 8 - on))
    node_out = _mlp3(x=xh, Ws=[dn["Ws"][0], dn["Ws"][1], w3n],
                     bs=[dn["bs"][0], dn["bs"][1], b3n], ln=None, blk=NBLK)
    de = params["dec_edge"]
    oe = de["Ws"][2].shape[1]
    w3e = jnp.pad(de["Ws"][2], ((0, 0), (0, 8 - oe)))
    b3e = jnp.pad(de["bs"][2], (0, 8 - oe))
    edge_out = _mlp3(x=e, Ws=[de["Ws"][0], de["Ws"][1], w3e],
                     bs=[de["bs"][0], de["bs"][1], b3e], ln=None, blk=EBLK)
    return (node_out[:, :on], edge_out[:E, :oe])
